# baseline (speedup 1.0000x reference)
import functools

import jax
import jax.numpy as jnp
from jax import lax
from jax.experimental import pallas as pl
from jax.experimental.pallas import tpu as pltpu
from jax.experimental.pallas import tpu_sc as plsc

N = 2097152
D = 16
NC = 2
NS = 16
NW = NC * NS
C = 1024
G = C // 128
RW = N // NW
NCHUNK = RW // C
U = 8

_mesh = plsc.VectorSubcoreMesh(core_axis_name="c", subcore_axis_name="s")


@functools.partial(
    pl.kernel,
    mesh=_mesh,
    compiler_params=pltpu.CompilerParams(use_tc_tiling_on_sc=False),
    out_type=jax.ShapeDtypeStruct((N, D), jnp.float32),
    scratch_types=[
        pltpu.VMEM((G, 128), jnp.int32),
        pltpu.VMEM((C, D), jnp.float32),
        pltpu.VMEM((C, D), jnp.float32),
        pltpu.VMEM((16,), jnp.float32),
        pltpu.SemaphoreType.DMA,
    ],
)
def _softshrink_sc(x_hbm, idx_hbm, t_hbm, out_hbm, idx_v, rows_v, x_v, t_v, sem):
    wid = lax.axis_index("s") * NC + lax.axis_index("c")
    base = wid * RW
    pltpu.sync_copy(t_hbm, t_v)
    tvec = t_v[...]

    def chunk_body(j, carry):
        off = base + j * C
        pltpu.sync_copy(idx_hbm.at[pl.ds(wid * (RW // 128) + j * G, G)], idx_v)
        copies = []
        for g in range(G):
            copies.append(
                pltpu.async_copy(
                    x_hbm.at[idx_v.at[g]],
                    rows_v.at[pl.ds(g * 128, 128)],
                    sem,
                )
            )
        pltpu.sync_copy(x_hbm.at[pl.ds(off, C)], x_v)
        for cp in copies:
            cp.wait()

        def row_body(i, carry2):
            ibase = i * U
            for u in range(U):
                r = ibase + u
                gv = rows_v[r]
                xv = x_v[r]
                rows_v[r] = jnp.maximum(jnp.abs(gv) - tvec, 0.0) * jnp.sign(xv)
            return carry2

        lax.fori_loop(0, C // U, row_body, 0)
        pltpu.sync_copy(rows_v, out_hbm.at[pl.ds(off, C)])
        return carry

    lax.fori_loop(0, NCHUNK, chunk_body, 0)


def kernel(x, rho, indices, thres):
    t = jax.nn.softplus(thres[0]) / rho[0]
    t16 = jnp.full((16,), t, dtype=jnp.float32)
    idx = indices.astype(jnp.int32).reshape(N // 128, 128)
    return _softshrink_sc(x, idx, t16)
